# baseline (speedup 1.0000x reference)
import jax
import jax.numpy as jnp
from jax import lax
from jax.experimental import pallas as pl

STEPS_PER_DAY = 288
TE_DIM = 32
NUM_NODES = 170
SE_DIM = 16
OUT_DIM = SE_DIM + TE_DIM
ND = NUM_NODES * OUT_DIM


def _body(tod_ref, dow_ref, se_ref, a_ref, tw_ref, dw_ref, o_ref):
    bsz = tod_ref.shape[-1]
    tod = jnp.clip(tod_ref[0], 0, STEPS_PER_DAY - 1)
    dow = jnp.clip(dow_ref[0], 0, 6)
    oh_t = (tod == lax.broadcasted_iota(jnp.int32, (STEPS_PER_DAY, bsz), 0))
    oh_d = (dow == lax.broadcasted_iota(jnp.int32, (8, bsz), 0))
    te_t = lax.dot(tw_ref[...], oh_t.astype(jnp.float32),
                   preferred_element_type=jnp.float32)
    te_t = te_t + lax.dot(dw_ref[...], oh_d.astype(jnp.float32),
                          preferred_element_type=jnp.float32)
    bmat = jnp.concatenate(
        [jnp.zeros((SE_DIM, bsz), jnp.float32), te_t], axis=0)
    o_ref[0] = lax.dot(a_ref[...], bmat,
                       preferred_element_type=jnp.float32) + se_ref[...]


@jax.jit
def kernel(te, se, tod_weight, dow_weight):
    b, t = te.shape[0], te.shape[1]
    tod_ids = te[..., 0].astype(jnp.int32).T.reshape(t, 1, b)
    dow_ids = te[..., 1].astype(jnp.int32).T.reshape(t, 1, b)
    tod_wT = tod_weight.T
    dow_wT = jnp.pad(dow_weight, ((0, 1), (0, 0))).T
    se_col = jnp.pad(se, ((0, 0), (0, TE_DIM))).reshape(ND, 1)
    amat = jnp.tile(jnp.eye(OUT_DIM, dtype=jnp.float32),
                    (NUM_NODES, 1))

    nsplit = 4
    nd_blk = ND // nsplit
    out = pl.pallas_call(
        _body,
        grid=(t, nsplit),
        in_specs=[
            pl.BlockSpec((1, 1, b), lambda i, j: (i, 0, 0)),
            pl.BlockSpec((1, 1, b), lambda i, j: (i, 0, 0)),
            pl.BlockSpec((nd_blk, 1), lambda i, j: (j, 0)),
            pl.BlockSpec((nd_blk, OUT_DIM), lambda i, j: (j, 0)),
            pl.BlockSpec((TE_DIM, STEPS_PER_DAY), lambda i, j: (0, 0)),
            pl.BlockSpec((TE_DIM, 8), lambda i, j: (0, 0)),
        ],
        out_specs=pl.BlockSpec((1, nd_blk, b), lambda i, j: (i, j, 0)),
        out_shape=jax.ShapeDtypeStruct((t, ND, b), jnp.float32),
    )(tod_ids, dow_ids, se_col, amat, tod_wT, dow_wT)
    out = out.reshape(t, NUM_NODES, OUT_DIM, b)
    return jnp.transpose(out, (3, 0, 1, 2))


# speedup vs baseline: 1.8628x; 1.8628x over previous
import jax
import jax.numpy as jnp
from jax import lax
from jax.experimental import pallas as pl
from jax.experimental.pallas import tpu as pltpu

STEPS_PER_DAY = 288
TE_DIM = 32
NUM_NODES = 170
SE_DIM = 16
OUT_DIM = SE_DIM + TE_DIM
ND = NUM_NODES * OUT_DIM
NCHUNK = 4
CH = ND // NCHUNK


def _body(tod_ref, dow_ref, se_ref, a_ref, tw_ref, dw_ref, o_ref, buf, sem):
    i = pl.program_id(0)
    nt = pl.num_programs(0)
    bsz = tod_ref.shape[-1]
    tod = jnp.clip(tod_ref[0], 0, STEPS_PER_DAY - 1)
    dow = jnp.clip(dow_ref[0], 0, 6)
    oh_t = (tod == lax.broadcasted_iota(jnp.int32, (STEPS_PER_DAY, bsz), 0))
    oh_d = (dow == lax.broadcasted_iota(jnp.int32, (8, bsz), 0))
    te_t = lax.dot(tw_ref[...], oh_t.astype(jnp.float32),
                   preferred_element_type=jnp.float32)
    te_t = te_t + lax.dot(dw_ref[...], oh_d.astype(jnp.float32),
                          preferred_element_type=jnp.float32)
    bmat = jnp.concatenate(
        [jnp.zeros((SE_DIM, bsz), jnp.float32), te_t], axis=0)
    res = lax.dot(a_ref[...], bmat,
                  preferred_element_type=jnp.float32) + se_ref[...]

    slot = lax.rem(i, 2)

    @pl.when(i >= 2)
    def _wait_prev():
        for k in range(NCHUNK):
            pltpu.make_async_copy(
                buf.at[slot, pl.ds(k * CH, CH), :],
                o_ref.at[i - 2, pl.ds(k * CH, CH), :],
                sem.at[slot, k]).wait()

    buf[slot] = res
    for k in range(NCHUNK):
        pltpu.make_async_copy(
            buf.at[slot, pl.ds(k * CH, CH), :],
            o_ref.at[i, pl.ds(k * CH, CH), :],
            sem.at[slot, k]).start()

    @pl.when(i == nt - 1)
    def _drain():
        for k in range(NCHUNK):
            pltpu.make_async_copy(
                buf.at[slot, pl.ds(k * CH, CH), :],
                o_ref.at[i, pl.ds(k * CH, CH), :],
                sem.at[slot, k]).wait()
            pltpu.make_async_copy(
                buf.at[1 - slot, pl.ds(k * CH, CH), :],
                o_ref.at[i, pl.ds(k * CH, CH), :],
                sem.at[1 - slot, k]).wait()


@jax.jit
def kernel(te, se, tod_weight, dow_weight):
    b, t = te.shape[0], te.shape[1]
    tod_ids = te[..., 0].astype(jnp.int32).T.reshape(t, 1, b)
    dow_ids = te[..., 1].astype(jnp.int32).T.reshape(t, 1, b)
    tod_wT = tod_weight.T
    dow_wT = jnp.pad(dow_weight, ((0, 1), (0, 0))).T
    se_col = jnp.pad(se, ((0, 0), (0, TE_DIM))).reshape(ND, 1)
    amat = jnp.tile(jnp.eye(OUT_DIM, dtype=jnp.float32),
                    (NUM_NODES, 1))

    out = pl.pallas_call(
        _body,
        grid=(t,),
        in_specs=[
            pl.BlockSpec((1, 1, b), lambda i: (i, 0, 0)),
            pl.BlockSpec((1, 1, b), lambda i: (i, 0, 0)),
            pl.BlockSpec((ND, 1), lambda i: (0, 0)),
            pl.BlockSpec((ND, OUT_DIM), lambda i: (0, 0)),
            pl.BlockSpec((TE_DIM, STEPS_PER_DAY), lambda i: (0, 0)),
            pl.BlockSpec((TE_DIM, 8), lambda i: (0, 0)),
        ],
        out_specs=pl.BlockSpec(memory_space=pl.ANY),
        out_shape=jax.ShapeDtypeStruct((t, ND, b), jnp.float32),
        scratch_shapes=[
            pltpu.VMEM((2, ND, b), jnp.float32),
            pltpu.SemaphoreType.DMA((2, NCHUNK)),
        ],
    )(tod_ids, dow_ids, se_col, amat, tod_wT, dow_wT)
    out = out.reshape(t, NUM_NODES, OUT_DIM, b)
    return jnp.transpose(out, (3, 0, 1, 2))
